# baseline (speedup 1.0000x reference)
import jax
import jax.numpy as jnp
from jax import lax
from jax.experimental import pallas as pl
from jax.experimental.pallas import tpu as pltpu
from jax.experimental.pallas import tpu_sc as plsc

N = 10000
E = 320000
D = 128

NC = 2
NS = 16
HD = D // NC
EPT = E // NS
CH = 80
NCHUNK = EPT // CH
HCHUNK = NCHUNK // 2
NBUF = 5
LA = 3
NI = 10
IA = 4
ROUNDS = NCHUNK // NI
CLAG = 3
TPX = N // NS
NP = 10240
TPN = NP // NS
SCH = 128
CW = 16


def _make_agg_body(with_counts):
    def body(x_hbm, edge_hbm, zrow_hbm, zcnt_hbm, ones_hbm,
             outp_hbm, outc_hbm, *scratch):
        rows = scratch[0:NBUF]
        iring = scratch[NBUF:NBUF + NI]
        ones_v, x_sh, agg_sh, cnt_sh = scratch[NBUF + NI:NBUF + NI + 4]
        k = NBUF + NI + 4
        gsem = scratch[k:k + NBUF]
        ssem = scratch[k + NBUF:k + 2 * NBUF]
        isem = scratch[k + 2 * NBUF:k + 2 * NBUF + NI]
        csem = scratch[k + 2 * NBUF + NI]

        cid = lax.axis_index("c")
        sid = lax.axis_index("s")
        base = pl.multiple_of(sid * TPN, 8)
        xbase = sid * TPX
        lo = cid * HCHUNK

        def gissue(e, b):
            pltpu.async_copy(x_sh.at[iring[e].at[0]], rows[b], gsem[b])

        if with_counts:
            pltpu.sync_copy(ones_hbm, ones_v)

        @pl.when(cid == 0)
        def _():
            pltpu.sync_copy(x_hbm.at[pl.ds(xbase, TPX), pl.ds(0, HD)],
                            x_sh.at[pl.ds(xbase, TPX)])

        @pl.when(cid == 1)
        def _():
            pltpu.sync_copy(x_hbm.at[pl.ds(xbase, TPX), pl.ds(HD, HD)],
                            x_sh.at[pl.ds(xbase, TPX)])
        pltpu.sync_copy(zrow_hbm, agg_sh.at[pl.ds(base, TPN)])
        if with_counts:
            pltpu.sync_copy(zcnt_hbm, cnt_sh.at[pl.ds(base, TPN)])
        plsc.subcore_barrier()

        ebase = sid * EPT

        def iload(e, c):
            off = ebase + c * CH
            pltpu.async_copy(edge_hbm.at[0, pl.ds(off, CH)], iring[e].at[0],
                             isem[e])
            pltpu.async_copy(edge_hbm.at[1, pl.ds(off, CH)], iring[e].at[1],
                             isem[e])

        def iwait(e):
            pltpu.make_async_copy(edge_hbm.at[0, pl.ds(0, CH)], iring[e].at[0],
                                  isem[e]).wait()
            pltpu.make_async_copy(edge_hbm.at[1, pl.ds(0, CH)], iring[e].at[1],
                                  isem[e]).wait()

        def gwait(b):
            pltpu.make_async_copy(x_sh.at[iring[0].at[0]], rows[b],
                                  gsem[b]).wait()

        def swait(b):
            pltpu.make_async_copy(rows[b], agg_sh.at[iring[0].at[1]],
                                  ssem[b]).wait()

        def cwait():
            pltpu.make_async_copy(ones_v, cnt_sh.at[iring[0].at[1]],
                                  csem).wait()

        for c in range(IA):
            iload(c, c)
        for b in range(LA):
            iwait(b)
            gissue(b, b)

        @pl.loop(0, ROUNDS)
        def _(g):
            for u in range(NI):
                j = g * NI + u
                bb = (u + LA) % NBUF
                e2 = (u + LA) % NI
                e3 = (u + IA) % NI
                jj = j + LA
                jjj = j + IA

                @pl.when(jjj < NCHUNK)
                def _():
                    iload(e3, jjj)

                @pl.when(jj >= NBUF)
                def _():
                    swait(bb)

                @pl.when(jj < NCHUNK)
                def _():
                    iwait(e2)
                    gissue(e2, bb)

                gwait(u % NBUF)
                pltpu.async_copy(rows[u % NBUF], agg_sh.at[iring[u].at[1]],
                                 ssem[u % NBUF], add=True)

                if with_counts:
                    @pl.when((j >= lo) & (j < lo + HCHUNK))
                    def _():
                        @pl.when(j >= lo + CLAG)
                        def _():
                            cwait()
                        pltpu.async_copy(ones_v, cnt_sh.at[iring[u].at[1]],
                                         csem, add=True)

        for kk in range(NCHUNK - (NBUF - LA), NCHUNK):
            swait(kk % NBUF)
        if with_counts:
            @pl.loop(0, CLAG)
            def _(t):
                cwait()
        plsc.subcore_barrier()

        @pl.when(cid == 0)
        def _():
            pltpu.sync_copy(agg_sh.at[pl.ds(base, TPN)],
                            outp_hbm.at[pl.ds(base, TPN), pl.ds(0, HD)])
            if with_counts:
                pltpu.sync_copy(cnt_sh.at[pl.ds(base, TPN)],
                                outc_hbm.at[pl.ds(base, TPN), pl.ds(0, CW)])

        @pl.when(cid == 1)
        def _():
            pltpu.sync_copy(agg_sh.at[pl.ds(base, TPN)],
                            outp_hbm.at[pl.ds(base, TPN), pl.ds(HD, HD)])
            if with_counts:
                pltpu.sync_copy(cnt_sh.at[pl.ds(base, TPN)],
                                outc_hbm.at[pl.ds(base, TPN), pl.ds(CW, CW)])

    return body


def _sc_aggregate(x, edge, zrow, zcnt, ones, with_counts):
    mesh = plsc.VectorSubcoreMesh(core_axis_name="c", subcore_axis_name="s")
    return pl.kernel(
        _make_agg_body(with_counts),
        out_type=[
            jax.ShapeDtypeStruct((NP, D), jnp.float32),
            jax.ShapeDtypeStruct((NP, D), jnp.float32),
        ],
        mesh=mesh,
        compiler_params=pltpu.CompilerParams(use_tc_tiling_on_sc=False),
        scratch_types=(
            [pltpu.VMEM((CH, HD), jnp.float32) for _ in range(NBUF)]
            + [pltpu.VMEM((2, CH), jnp.int32) for _ in range(NI)]
            + [
                pltpu.VMEM((CH, CW), jnp.float32),
                pltpu.VMEM_SHARED((N, HD), jnp.float32),
                pltpu.VMEM_SHARED((NP, HD), jnp.float32),
                pltpu.VMEM_SHARED((NP, CW), jnp.float32),
            ]
            + [pltpu.SemaphoreType.DMA for _ in range(2 * NBUF + NI + 1)]
        ),
    )(x, edge, zrow, zcnt, ones)


ROWS_BLK = 2000


def _combine_body(p_ref, c_ref, x_ref, wlT_ref, wrT_ref, b_ref, o_ref):
    cnt = c_ref[:, 0:1] + c_ref[:, CW:CW + 1]
    recip = 1.0 / jnp.maximum(cnt, 1.0)
    mean = p_ref[...] * recip
    h = (jnp.dot(mean, wlT_ref[...], preferred_element_type=jnp.float32)
         + jnp.dot(x_ref[...], wrT_ref[...], preferred_element_type=jnp.float32)
         + b_ref[...])
    o_ref[...] = jnp.maximum(h, 0.0)


def _tc_combine(p, c, x, wlT, wrT, b):
    grid = (N // ROWS_BLK,)
    return pl.pallas_call(
        _combine_body,
        grid=grid,
        in_specs=[
            pl.BlockSpec((ROWS_BLK, D), lambda i: (i, 0)),
            pl.BlockSpec((ROWS_BLK, D), lambda i: (i, 0)),
            pl.BlockSpec((ROWS_BLK, D), lambda i: (i, 0)),
            pl.BlockSpec((D, D), lambda i: (0, 0)),
            pl.BlockSpec((D, D), lambda i: (0, 0)),
            pl.BlockSpec((1, D), lambda i: (0, 0)),
        ],
        out_specs=pl.BlockSpec((ROWS_BLK, D), lambda i: (i, 0)),
        out_shape=jax.ShapeDtypeStruct((N, D), jnp.float32),
    )(p, c, x, wlT, wrT, b)


def kernel(x, edge_index, Wl1, Wr1, b1, Wl2, Wr2, b2):
    edge = edge_index.astype(jnp.int32)
    zrow = jnp.zeros((TPN, HD), jnp.float32)
    zcnt = jnp.zeros((TPN, CW), jnp.float32)
    ones = jnp.ones((CH, CW), jnp.float32)

    p1, c1 = _sc_aggregate(x, edge, zrow, zcnt, ones, True)
    h1 = _tc_combine(p1, c1, x, Wl1.T, Wr1.T, b1.reshape(1, D))
    p2, _ = _sc_aggregate(h1, edge, zrow, zcnt, ones, False)
    h2 = _tc_combine(p2, c1, h1, Wl2.T, Wr2.T, b2.reshape(1, D))
    return h2


# speedup vs baseline: 1.0026x; 1.0026x over previous
import jax
import jax.numpy as jnp
from jax import lax
from jax.experimental import pallas as pl
from jax.experimental.pallas import tpu as pltpu
from jax.experimental.pallas import tpu_sc as plsc

N = 10000
E = 320000
D = 128

NC = 2
NS = 16
HD = D // NC
EPT = E // NS
CH = 80
NCHUNK = EPT // CH
HCHUNK = NCHUNK // 2
NBUF = 5
LA = 2
NI = 10
IA = 4
ROUNDS = NCHUNK // NI
CLAG = 3
TPX = N // NS
NP = 10240
TPN = NP // NS
SCH = 128
CW = 16


def _make_agg_body(with_counts):
    def body(x_hbm, edge_hbm, zrow_hbm, zcnt_hbm, ones_hbm,
             outp_hbm, outc_hbm, *scratch):
        rows = scratch[0:NBUF]
        iring = scratch[NBUF:NBUF + NI]
        ones_v, x_sh, agg_sh, cnt_sh = scratch[NBUF + NI:NBUF + NI + 4]
        k = NBUF + NI + 4
        gsem = scratch[k:k + NBUF]
        ssem = scratch[k + NBUF:k + 2 * NBUF]
        isem = scratch[k + 2 * NBUF:k + 2 * NBUF + NI]
        csem = scratch[k + 2 * NBUF + NI]

        cid = lax.axis_index("c")
        sid = lax.axis_index("s")
        base = pl.multiple_of(sid * TPN, 8)
        xbase = sid * TPX
        lo = cid * HCHUNK

        def gissue(e, b):
            pltpu.async_copy(x_sh.at[iring[e].at[0]], rows[b], gsem[b])

        if with_counts:
            pltpu.sync_copy(ones_hbm, ones_v)

        @pl.when(cid == 0)
        def _():
            pltpu.sync_copy(x_hbm.at[pl.ds(xbase, TPX), pl.ds(0, HD)],
                            x_sh.at[pl.ds(xbase, TPX)])

        @pl.when(cid == 1)
        def _():
            pltpu.sync_copy(x_hbm.at[pl.ds(xbase, TPX), pl.ds(HD, HD)],
                            x_sh.at[pl.ds(xbase, TPX)])
        pltpu.sync_copy(zrow_hbm, agg_sh.at[pl.ds(base, TPN)])
        if with_counts:
            pltpu.sync_copy(zcnt_hbm, cnt_sh.at[pl.ds(base, TPN)])
        plsc.subcore_barrier()

        ebase = sid * EPT

        def iload(e, c):
            off = ebase + c * CH
            pltpu.async_copy(edge_hbm.at[0, pl.ds(off, CH)], iring[e].at[0],
                             isem[e])
            pltpu.async_copy(edge_hbm.at[1, pl.ds(off, CH)], iring[e].at[1],
                             isem[e])

        def iwait(e):
            pltpu.make_async_copy(edge_hbm.at[0, pl.ds(0, CH)], iring[e].at[0],
                                  isem[e]).wait()
            pltpu.make_async_copy(edge_hbm.at[1, pl.ds(0, CH)], iring[e].at[1],
                                  isem[e]).wait()

        def gwait(b):
            pltpu.make_async_copy(x_sh.at[iring[0].at[0]], rows[b],
                                  gsem[b]).wait()

        def swait(b):
            pltpu.make_async_copy(rows[b], agg_sh.at[iring[0].at[1]],
                                  ssem[b]).wait()

        def cwait():
            pltpu.make_async_copy(ones_v, cnt_sh.at[iring[0].at[1]],
                                  csem).wait()

        for c in range(IA):
            iload(c, c)
        for b in range(LA):
            iwait(b)
            gissue(b, b)

        @pl.loop(0, ROUNDS)
        def _(g):
            for u in range(NI):
                j = g * NI + u
                bb = (u + LA) % NBUF
                e2 = (u + LA) % NI
                e3 = (u + IA) % NI
                jj = j + LA
                jjj = j + IA

                @pl.when(jjj < NCHUNK)
                def _():
                    iload(e3, jjj)

                @pl.when(jj >= NBUF)
                def _():
                    swait(bb)

                @pl.when(jj < NCHUNK)
                def _():
                    iwait(e2)
                    gissue(e2, bb)

                gwait(u % NBUF)
                pltpu.async_copy(rows[u % NBUF], agg_sh.at[iring[u].at[1]],
                                 ssem[u % NBUF], add=True)

                if with_counts:
                    @pl.when((j >= lo) & (j < lo + HCHUNK))
                    def _():
                        @pl.when(j >= lo + CLAG)
                        def _():
                            cwait()
                        pltpu.async_copy(ones_v, cnt_sh.at[iring[u].at[1]],
                                         csem, add=True)

        for kk in range(NCHUNK - (NBUF - LA), NCHUNK):
            swait(kk % NBUF)
        if with_counts:
            @pl.loop(0, CLAG)
            def _(t):
                cwait()
        plsc.subcore_barrier()

        @pl.when(cid == 0)
        def _():
            pltpu.sync_copy(agg_sh.at[pl.ds(base, TPN)],
                            outp_hbm.at[pl.ds(base, TPN), pl.ds(0, HD)])
            if with_counts:
                pltpu.sync_copy(cnt_sh.at[pl.ds(base, TPN)],
                                outc_hbm.at[pl.ds(base, TPN), pl.ds(0, CW)])

        @pl.when(cid == 1)
        def _():
            pltpu.sync_copy(agg_sh.at[pl.ds(base, TPN)],
                            outp_hbm.at[pl.ds(base, TPN), pl.ds(HD, HD)])
            if with_counts:
                pltpu.sync_copy(cnt_sh.at[pl.ds(base, TPN)],
                                outc_hbm.at[pl.ds(base, TPN), pl.ds(CW, CW)])

    return body


def _sc_aggregate(x, edge, zrow, zcnt, ones, with_counts):
    mesh = plsc.VectorSubcoreMesh(core_axis_name="c", subcore_axis_name="s")
    return pl.kernel(
        _make_agg_body(with_counts),
        out_type=[
            jax.ShapeDtypeStruct((NP, D), jnp.float32),
            jax.ShapeDtypeStruct((NP, D), jnp.float32),
        ],
        mesh=mesh,
        compiler_params=pltpu.CompilerParams(use_tc_tiling_on_sc=False),
        scratch_types=(
            [pltpu.VMEM((CH, HD), jnp.float32) for _ in range(NBUF)]
            + [pltpu.VMEM((2, CH), jnp.int32) for _ in range(NI)]
            + [
                pltpu.VMEM((CH, CW), jnp.float32),
                pltpu.VMEM_SHARED((N, HD), jnp.float32),
                pltpu.VMEM_SHARED((NP, HD), jnp.float32),
                pltpu.VMEM_SHARED((NP, CW), jnp.float32),
            ]
            + [pltpu.SemaphoreType.DMA for _ in range(2 * NBUF + NI + 1)]
        ),
    )(x, edge, zrow, zcnt, ones)


ROWS_BLK = 2000


def _combine_body(p_ref, c_ref, x_ref, wlT_ref, wrT_ref, b_ref, o_ref):
    cnt = c_ref[:, 0:1] + c_ref[:, CW:CW + 1]
    recip = 1.0 / jnp.maximum(cnt, 1.0)
    mean = p_ref[...] * recip
    h = (jnp.dot(mean, wlT_ref[...], preferred_element_type=jnp.float32)
         + jnp.dot(x_ref[...], wrT_ref[...], preferred_element_type=jnp.float32)
         + b_ref[...])
    o_ref[...] = jnp.maximum(h, 0.0)


def _tc_combine(p, c, x, wlT, wrT, b):
    grid = (N // ROWS_BLK,)
    return pl.pallas_call(
        _combine_body,
        grid=grid,
        in_specs=[
            pl.BlockSpec((ROWS_BLK, D), lambda i: (i, 0)),
            pl.BlockSpec((ROWS_BLK, D), lambda i: (i, 0)),
            pl.BlockSpec((ROWS_BLK, D), lambda i: (i, 0)),
            pl.BlockSpec((D, D), lambda i: (0, 0)),
            pl.BlockSpec((D, D), lambda i: (0, 0)),
            pl.BlockSpec((1, D), lambda i: (0, 0)),
        ],
        out_specs=pl.BlockSpec((ROWS_BLK, D), lambda i: (i, 0)),
        out_shape=jax.ShapeDtypeStruct((N, D), jnp.float32),
    )(p, c, x, wlT, wrT, b)


def kernel(x, edge_index, Wl1, Wr1, b1, Wl2, Wr2, b2):
    edge = edge_index.astype(jnp.int32)
    zrow = jnp.zeros((TPN, HD), jnp.float32)
    zcnt = jnp.zeros((TPN, CW), jnp.float32)
    ones = jnp.ones((CH, CW), jnp.float32)

    p1, c1 = _sc_aggregate(x, edge, zrow, zcnt, ones, True)
    h1 = _tc_combine(p1, c1, x, Wl1.T, Wr1.T, b1.reshape(1, D))
    p2, _ = _sc_aggregate(h1, edge, zrow, zcnt, ones, False)
    h2 = _tc_combine(p2, c1, h1, Wl2.T, Wr2.T, b2.reshape(1, D))
    return h2
